# Initial kernel scaffold; baseline (speedup 1.0000x reference)
import jax
import jax.numpy as jnp
from jax.experimental import pallas as pl


def kernel(ids, emb_weight):
    raise NotImplementedError("write your pallas kernel here")


# speedup vs baseline: 1.7798x; 1.7798x over previous
import jax
import jax.numpy as jnp
from jax import lax
from jax.experimental import pallas as pl
from jax.experimental.pallas import tpu as pltpu
from jax.experimental.pallas import tpu_sc as plsc

DIM = 768
BATCH = 16384
SEQ = 16

NC = 2
NS = 16
LANES = 16
NW = NC * NS
BPW = BATCH // NW
ROWS = 4
GROWS = ROWS * SEQ
NCHUNK = BPW // ROWS
DSLICES = DIM // LANES


def _embed_mean_body(ids_hbm, table_hbm, out_hbm,
                     idx_v, g0, g1, ob0, ob1, gsem0, gsem1):
    wid = lax.axis_index("s") * NC + lax.axis_index("c")
    wbase = wid * BPW

    pltpu.sync_copy(ids_hbm.at[pl.ds(wbase * SEQ, BPW * SEQ)], idx_v)

    gbufs = (g0, g1)
    obufs = (ob0, ob1)
    gsems = (gsem0, gsem1)

    def _fire(c, b):
        idx = idx_v.at[pl.ds(c * GROWS, GROWS)]
        pltpu.async_copy(table_hbm.at[idx], gbufs[b], gsems[b])

    def _wait(b):
        idx = idx_v.at[pl.ds(0, GROWS)]
        pltpu.make_async_copy(table_hbm.at[idx], gbufs[b], gsems[b]).wait()

    def _reduce(b):
        gb, ob = gbufs[b], obufs[b]
        for rr in range(ROWS):
            def dbody(d, _, rr=rr):
                sl = pl.ds(d * LANES, LANES)
                v = [gb[rr * SEQ + j, sl] for j in range(SEQ)]
                while len(v) > 1:
                    v = [v[i] + v[i + 1] for i in range(0, len(v), 2)]
                ob[rr, sl] = v[0] * (1.0 / SEQ)
                return 0
            lax.fori_loop(0, DSLICES, dbody, 0)

    def _out(c, b):
        pltpu.sync_copy(obufs[b], out_hbm.at[pl.ds(wbase + c * ROWS, ROWS)])

    _fire(0, 0)

    def loop_body(i, _):
        g = 2 * i
        _fire(g + 1, 1)
        _wait(0)
        _reduce(0)
        _out(g, 0)
        _fire(g + 2, 0)
        _wait(1)
        _reduce(1)
        _out(g + 1, 1)
        return 0

    lax.fori_loop(0, NCHUNK // 2 - 1, loop_body, 0)

    _fire(NCHUNK - 1, 1)
    _wait(0)
    _reduce(0)
    _out(NCHUNK - 2, 0)
    _wait(1)
    _reduce(1)
    _out(NCHUNK - 1, 1)


@jax.jit
def _embed_mean(ids_flat, emb_weight):
    mesh = plsc.VectorSubcoreMesh(
        core_axis_name="c", subcore_axis_name="s",
        num_cores=NC, num_subcores=NS)
    return pl.kernel(
        _embed_mean_body,
        out_type=jax.ShapeDtypeStruct((BATCH, DIM), jnp.float32),
        mesh=mesh,
        scratch_types=[
            pltpu.VMEM((BPW * SEQ,), jnp.int32),
            pltpu.VMEM((GROWS, DIM), jnp.float32),
            pltpu.VMEM((GROWS, DIM), jnp.float32),
            pltpu.VMEM((ROWS, DIM), jnp.float32),
            pltpu.VMEM((ROWS, DIM), jnp.float32),
            pltpu.SemaphoreType.DMA,
            pltpu.SemaphoreType.DMA,
        ],
    )(ids_flat, emb_weight)


def kernel(ids, emb_weight):
    ids_flat = jnp.reshape(ids, (-1,)).astype(jnp.int32)
    return _embed_mean(ids_flat, emb_weight)


# speedup vs baseline: 2.4875x; 1.3976x over previous
import jax
import jax.numpy as jnp
from jax import lax
from jax.experimental import pallas as pl
from jax.experimental.pallas import tpu as pltpu
from jax.experimental.pallas import tpu_sc as plsc

DIM = 768
BATCH = 16384
SEQ = 16

NC = 2
NS = 16
LANES = 16
NW = NC * NS
BPW = BATCH // NW
ROWS = 4
GROWS = ROWS * SEQ
NCHUNK = BPW // ROWS
DSLICES = DIM // LANES


def _embed_mean_body(ids_hbm, table_hbm, out_hbm,
                     idx_v, g0, g1, ob0, ob1, gsem0, gsem1):
    wid = lax.axis_index("s") * NC + lax.axis_index("c")
    wbase = wid * BPW

    pltpu.sync_copy(ids_hbm.at[pl.ds(wbase * SEQ, BPW * SEQ)], idx_v)

    gbufs = (g0, g1)
    obufs = (ob0, ob1)
    gsems = (gsem0, gsem1)

    def _fire(c, b):
        idx = idx_v.at[pl.ds(c * GROWS, GROWS)]
        pltpu.async_copy(table_hbm.at[idx], gbufs[b], gsems[b])

    def _wait(b):
        idx = idx_v.at[pl.ds(0, GROWS)]
        pltpu.make_async_copy(table_hbm.at[idx], gbufs[b], gsems[b]).wait()

    def _reduce(b):
        gb, ob = gbufs[b], obufs[b]
        for rr in range(ROWS):
            @plsc.parallel_loop(0, DSLICES, 1, unroll=4)
            def _(d, rr=rr):
                sl = pl.ds(d * LANES, LANES)
                v = [gb[rr * SEQ + j, sl] for j in range(SEQ)]
                while len(v) > 1:
                    v = [v[i] + v[i + 1] for i in range(0, len(v), 2)]
                ob[rr, sl] = v[0] * (1.0 / SEQ)

    def _out(c, b):
        pltpu.sync_copy(obufs[b], out_hbm.at[pl.ds(wbase + c * ROWS, ROWS)])

    _fire(0, 0)

    def loop_body(i, _):
        g = 2 * i
        _fire(g + 1, 1)
        _wait(0)
        _reduce(0)
        _out(g, 0)
        _fire(g + 2, 0)
        _wait(1)
        _reduce(1)
        _out(g + 1, 1)
        return 0

    lax.fori_loop(0, NCHUNK // 2 - 1, loop_body, 0)

    _fire(NCHUNK - 1, 1)
    _wait(0)
    _reduce(0)
    _out(NCHUNK - 2, 0)
    _wait(1)
    _reduce(1)
    _out(NCHUNK - 1, 1)


@jax.jit
def _embed_mean(ids_flat, emb_weight):
    mesh = plsc.VectorSubcoreMesh(
        core_axis_name="c", subcore_axis_name="s",
        num_cores=NC, num_subcores=NS)
    return pl.kernel(
        _embed_mean_body,
        out_type=jax.ShapeDtypeStruct((BATCH, DIM), jnp.float32),
        mesh=mesh,
        scratch_types=[
            pltpu.VMEM((BPW * SEQ,), jnp.int32),
            pltpu.VMEM((GROWS, DIM), jnp.float32),
            pltpu.VMEM((GROWS, DIM), jnp.float32),
            pltpu.VMEM((ROWS, DIM), jnp.float32),
            pltpu.VMEM((ROWS, DIM), jnp.float32),
            pltpu.SemaphoreType.DMA,
            pltpu.SemaphoreType.DMA,
        ],
    )(ids_flat, emb_weight)


def kernel(ids, emb_weight):
    ids_flat = jnp.reshape(ids, (-1,)).astype(jnp.int32)
    return _embed_mean(ids_flat, emb_weight)


# speedup vs baseline: 2.5656x; 1.0314x over previous
import jax
import jax.numpy as jnp
from jax import lax
from jax.experimental import pallas as pl
from jax.experimental.pallas import tpu as pltpu
from jax.experimental.pallas import tpu_sc as plsc

DIM = 768
BATCH = 16384
SEQ = 16

NC = 2
NS = 16
LANES = 16
NW = NC * NS
BPW = BATCH // NW
ROWS = 4
GROWS = ROWS * SEQ
NCHUNK = BPW // ROWS
DSLICES = DIM // LANES


def _embed_mean_body(ids_hbm, table_hbm, out_hbm,
                     idx_v, g0, g1, ob0, ob1, gsem0, gsem1, osem0, osem1):
    wid = lax.axis_index("s") * NC + lax.axis_index("c")
    wbase = wid * BPW

    pltpu.sync_copy(ids_hbm.at[pl.ds(wbase * SEQ, BPW * SEQ)], idx_v)

    gbufs = (g0, g1)
    obufs = (ob0, ob1)
    gsems = (gsem0, gsem1)
    osems = (osem0, osem1)

    def _fire(c, b):
        idx = idx_v.at[pl.ds(c * GROWS, GROWS)]
        pltpu.async_copy(table_hbm.at[idx], gbufs[b], gsems[b])

    def _wait(b):
        idx = idx_v.at[pl.ds(0, GROWS)]
        pltpu.make_async_copy(table_hbm.at[idx], gbufs[b], gsems[b]).wait()

    def _reduce(b):
        gb, ob = gbufs[b], obufs[b]
        for rr in range(ROWS):
            @plsc.parallel_loop(0, DSLICES, 1, unroll=4)
            def _(d, rr=rr):
                sl = pl.ds(d * LANES, LANES)
                v = [gb[rr * SEQ + j, sl] for j in range(SEQ)]
                while len(v) > 1:
                    v = [v[i] + v[i + 1] for i in range(0, len(v), 2)]
                ob[rr, sl] = v[0] * (1.0 / SEQ)

    def _out_fire(c, b):
        pltpu.async_copy(obufs[b], out_hbm.at[pl.ds(wbase + c * ROWS, ROWS)],
                         osems[b])

    def _out_wait(b):
        pltpu.make_async_copy(out_hbm.at[pl.ds(wbase, ROWS)], obufs[b],
                              osems[b]).wait()

    _fire(0, 0)
    pltpu.async_copy(out_hbm.at[pl.ds(wbase, ROWS)], ob0, osem0)
    pltpu.async_copy(out_hbm.at[pl.ds(wbase, ROWS)], ob1, osem1)

    def _step(c, b):
        _wait(b)
        _out_wait(b)
        _reduce(b)
        _out_fire(c, b)

    def loop_body(i, _):
        g = 2 * i
        _fire(g + 1, 1)
        _step(g, 0)
        _fire(g + 2, 0)
        _step(g + 1, 1)
        return 0

    lax.fori_loop(0, NCHUNK // 2 - 1, loop_body, 0)

    _fire(NCHUNK - 1, 1)
    _step(NCHUNK - 2, 0)
    _step(NCHUNK - 1, 1)
    _out_wait(0)
    _out_wait(1)


@jax.jit
def _embed_mean(ids_flat, emb_weight):
    mesh = plsc.VectorSubcoreMesh(
        core_axis_name="c", subcore_axis_name="s",
        num_cores=NC, num_subcores=NS)
    return pl.kernel(
        _embed_mean_body,
        out_type=jax.ShapeDtypeStruct((BATCH, DIM), jnp.float32),
        mesh=mesh,
        scratch_types=[
            pltpu.VMEM((BPW * SEQ,), jnp.int32),
            pltpu.VMEM((GROWS, DIM), jnp.float32),
            pltpu.VMEM((GROWS, DIM), jnp.float32),
            pltpu.VMEM((ROWS, DIM), jnp.float32),
            pltpu.VMEM((ROWS, DIM), jnp.float32),
            pltpu.SemaphoreType.DMA,
            pltpu.SemaphoreType.DMA,
            pltpu.SemaphoreType.DMA,
            pltpu.SemaphoreType.DMA,
        ],
    )(ids_flat, emb_weight)


def kernel(ids, emb_weight):
    ids_flat = jnp.reshape(ids, (-1,)).astype(jnp.int32)
    return _embed_mean(ids_flat, emb_weight)
